# Initial kernel scaffold; baseline (speedup 1.0000x reference)
import jax
import jax.numpy as jnp
from jax.experimental import pallas as pl


def kernel(x, edge_index, W, b, gamma, beta):
    raise NotImplementedError("write your pallas kernel here")


# speedup vs baseline: 22.0665x; 22.0665x over previous
import functools

import jax
import jax.numpy as jnp
from jax import lax
from jax.experimental import pallas as pl
from jax.experimental.pallas import tpu as pltpu
from jax.experimental.pallas import tpu_sc as plsc

NC = 2
NS = 16
L = 16
NW = NC * NS
CH = 128


def _degree_fn(n_pad, n_chunks):
  rpt = n_pad // NS
  mesh = plsc.VectorSubcoreMesh(core_axis_name="c", subcore_axis_name="s", num_cores=NC, num_subcores=NS)

  @functools.partial(
      pl.kernel,
      out_type=jax.ShapeDtypeStruct((NC, n_pad, L), jnp.float32),
      mesh=mesh,
      scratch_types=[
          pltpu.VMEM((n_chunks, CH), jnp.int32),
          pltpu.VMEM((CH, L), jnp.float32),
          pltpu.VMEM_SHARED((n_pad, L), jnp.float32),
      ],
      compiler_params=pltpu.CompilerParams(use_tc_tiling_on_sc=False),
  )
  def deg_kernel(dst3, ones_rows, zeros_rows, out, idx_v, ones_v, hist_sh):
    c = lax.axis_index("c")
    s = lax.axis_index("s")
    w = c * NS + s
    sl = pl.ds(s * rpt, rpt)
    pltpu.sync_copy(dst3.at[w], idx_v)
    pltpu.sync_copy(ones_rows, ones_v)
    pltpu.sync_copy(zeros_rows, hist_sh.at[sl])
    plsc.subcore_barrier()

    def body(j):
      pltpu.sync_copy(ones_v, hist_sh.at[idx_v.at[j]], add=True)

    pl.loop(0, n_chunks)(body)

    plsc.subcore_barrier()
    pltpu.sync_copy(hist_sh.at[sl], out.at[c, sl])

  return deg_kernel


def _gather_scatter_fn(n_pad, n_chunks, dh):
  rpt = n_pad // NS
  half = n_chunks // 2
  mesh = plsc.VectorSubcoreMesh(core_axis_name="c", subcore_axis_name="s", num_cores=NC, num_subcores=NS)

  @functools.partial(
      pl.kernel,
      out_type=jax.ShapeDtypeStruct((NC, n_pad, dh), jnp.float32),
      mesh=mesh,
      scratch_types=[
          pltpu.VMEM((n_chunks, CH), jnp.int32),
          pltpu.VMEM((n_chunks, CH), jnp.int32),
          pltpu.VMEM((CH, dh), jnp.float32),
          pltpu.VMEM((CH, dh), jnp.float32),
          pltpu.SemaphoreType.DMA,
          pltpu.SemaphoreType.DMA,
          pltpu.VMEM_SHARED((n_pad, dh), jnp.float32),
      ],
      compiler_params=pltpu.CompilerParams(use_tc_tiling_on_sc=False),
  )
  def gs_kernel(src3a, src3b, dst3, xwp, out,
                sidx, didx, buf0, buf1, sem0, sem1, agg_sh):
    c = lax.axis_index("c")
    s = lax.axis_index("s")
    sl = pl.ds(s * rpt, rpt)

    @pl.when(c == 0)
    def _():
      pltpu.sync_copy(src3a.at[s], sidx)

    @pl.when(c != 0)
    def _():
      pltpu.sync_copy(src3b.at[s], sidx)

    pltpu.sync_copy(dst3.at[s], didx)
    pltpu.sync_copy(xwp.at[pl.ds(c * n_pad + s * rpt, rpt)], agg_sh.at[sl])
    plsc.subcore_barrier()

    pltpu.async_copy(xwp.at[sidx.at[0]], buf0, sem0)

    def body(j):
      k = 2 * j
      pltpu.make_async_copy(xwp.at[sidx.at[k]], buf0, sem0).wait()
      pltpu.async_copy(xwp.at[sidx.at[k + 1]], buf1, sem1)
      pltpu.sync_copy(buf0, agg_sh.at[didx.at[k]], add=True)
      pltpu.make_async_copy(xwp.at[sidx.at[k + 1]], buf1, sem1).wait()

      @pl.when(j < half - 1)
      def _():
        pltpu.async_copy(xwp.at[sidx.at[k + 2]], buf0, sem0)

      pltpu.sync_copy(buf1, agg_sh.at[didx.at[k + 1]], add=True)

    pl.loop(0, half)(body)

    plsc.subcore_barrier()
    pltpu.sync_copy(agg_sh.at[sl], out.at[c, sl])

  return gs_kernel


def _mm_body(x_ref, w_ref, hist_ref, xwp_ref, dinv_ref):
  deg = hist_ref[0, :, 0:1] + hist_ref[1, :, 0:1] + 1.0
  dinv = lax.rsqrt(deg)
  xw = jnp.dot(x_ref[...], w_ref[...], preferred_element_type=jnp.float32)
  xws = xw * dinv
  dh = xw.shape[1] // NC
  xwp_ref[0] = xws[:, :dh]
  xwp_ref[1] = xws[:, dh:]
  dinv_ref[...] = dinv


def _ln_body(p_ref, dinv_ref, b_ref, g_ref, be_ref, o_ref):
  h = jnp.concatenate([p_ref[0], p_ref[1]], axis=-1)
  h = h * dinv_ref[...] + b_ref[...]
  mu = jnp.mean(h, axis=-1, keepdims=True)
  dlt = h - mu
  var = jnp.mean(dlt * dlt, axis=-1, keepdims=True)
  o_ref[...] = jnp.maximum(
      dlt * lax.rsqrt(var + 1e-5) * g_ref[...] + be_ref[...], 0.0)


def kernel(x, edge_index, W, b, gamma, beta):
  n = x.shape[0]
  e = edge_index.shape[1]
  d = x.shape[1]
  dh = d // NC

  n_pad = ((n + 8 * NS - 1) // (8 * NS)) * (8 * NS)
  per_round = NS * CH
  n_chunks = -(-e // per_round)
  n_chunks += n_chunks % 2
  e_pad = n_chunks * per_round
  rpt = n_pad // NS

  src = edge_index[0].astype(jnp.int32)
  dst = edge_index[1].astype(jnp.int32)
  src_p = jnp.concatenate([src, jnp.zeros((e_pad - e,), jnp.int32)])
  dst_p = jnp.concatenate([dst, jnp.full((e_pad - e,), n, jnp.int32)])
  src3a = src_p.reshape(NS, n_chunks, CH)
  src3b = src3a + jnp.int32(n_pad)
  dst3 = dst_p.reshape(NS, n_chunks, CH)
  x_p = jnp.pad(x, ((0, n_pad - n), (0, 0)))

  hist = _degree_fn(n_pad, n_chunks // NC)(
      dst_p.reshape(NW, n_chunks // NC, CH),
      jnp.ones((CH, L), jnp.float32),
      jnp.zeros((rpt, L), jnp.float32))

  bn = n_pad // 8 if (n_pad // 8) % 8 == 0 else n_pad
  grid = n_pad // bn
  xwp2, dinv = pl.pallas_call(
      _mm_body,
      grid=(grid,),
      in_specs=[
          pl.BlockSpec((bn, d), lambda i: (i, 0)),
          pl.BlockSpec((d, d), lambda i: (0, 0)),
          pl.BlockSpec((NC, bn, L), lambda i: (0, i, 0)),
      ],
      out_specs=[
          pl.BlockSpec((NC, bn, dh), lambda i: (0, i, 0)),
          pl.BlockSpec((bn, 1), lambda i: (i, 0)),
      ],
      out_shape=[
          jax.ShapeDtypeStruct((NC, n_pad, dh), jnp.float32),
          jax.ShapeDtypeStruct((n_pad, 1), jnp.float32),
      ],
  )(x_p, W, hist)

  partials = _gather_scatter_fn(n_pad, n_chunks, dh)(
      src3a, src3b, dst3, xwp2.reshape(NC * n_pad, dh))

  out = pl.pallas_call(
      _ln_body,
      grid=(grid,),
      in_specs=[
          pl.BlockSpec((NC, bn, dh), lambda i: (0, i, 0)),
          pl.BlockSpec((bn, 1), lambda i: (i, 0)),
          pl.BlockSpec((1, d), lambda i: (0, 0)),
          pl.BlockSpec((1, d), lambda i: (0, 0)),
          pl.BlockSpec((1, d), lambda i: (0, 0)),
      ],
      out_specs=pl.BlockSpec((bn, d), lambda i: (i, 0)),
      out_shape=jax.ShapeDtypeStruct((n_pad, d), jnp.float32),
  )(partials, dinv, b.reshape(1, d), gamma.reshape(1, d), beta.reshape(1, d))

  return out[:n]
